# Initial kernel scaffold; baseline (speedup 1.0000x reference)
import jax
import jax.numpy as jnp
from jax.experimental import pallas as pl


def kernel(anchors, rpn_cls_prob, rpn_bbox_pred, rpn_trans_param, im_info):
    raise NotImplementedError("write your pallas kernel here")


# speedup vs baseline: 154.6936x; 154.6936x over previous
import functools

import jax
import jax.numpy as jnp
from jax import lax
from jax.experimental import pallas as pl
from jax.experimental.pallas import tpu as pltpu

_NUM_ANCHORS = 9
_PRE = 6000
_POST = 300
_THRESH = 0.7
_B = 128
_NB = 48


def _nms_kernel(x1_ref, y1_ref, x2_ref, y2_ref, area_ref, keep_ref):
  f32 = jnp.float32
  ri = lax.broadcasted_iota(jnp.int32, (_NB, _B), 0)
  ci = lax.broadcasted_iota(jnp.int32, (_NB, _B), 1)
  keep_ref[...] = jnp.where(ri * _B + ci < _PRE, 1.0, 0.0).astype(f32)

  rr = lax.broadcasted_iota(jnp.int32, (_B, _B), 0)
  cc = lax.broadcasted_iota(jnp.int32, (_B, _B), 1)
  ident = (rr == cc).astype(f32)
  lower = (rr < cc).astype(f32)

  def _t(row):
    return lax.dot_general(ident, row, (((1,), (1,)), ((), ())),
                           preferred_element_type=f32)

  def _ov_mask(c_x1, c_y1, c_x2, c_y2, c_a, r_x1, r_y1, r_x2, r_y2, r_a):
    xx1 = jnp.maximum(c_x1, r_x1)
    yy1 = jnp.maximum(c_y1, r_y1)
    xx2 = jnp.minimum(c_x2, r_x2)
    yy2 = jnp.minimum(c_y2, r_y2)
    w = jnp.maximum(0.0, xx2 - xx1 + 1.0)
    h = jnp.maximum(0.0, yy2 - yy1 + 1.0)
    inter = w * h
    iou = inter / (c_a + r_a - inter)
    return (iou > _THRESH).astype(f32)

  def outer(i, carry):
    x1i = x1_ref[pl.ds(i, 1), :]
    y1i = y1_ref[pl.ds(i, 1), :]
    x2i = x2_ref[pl.ds(i, 1), :]
    y2i = y2_ref[pl.ds(i, 1), :]
    ai = area_ref[pl.ds(i, 1), :]
    cx1, cy1, cx2, cy2, ca = _t(x1i), _t(y1i), _t(x2i), _t(y2i), _t(ai)

    prior = keep_ref[pl.ds(i, 1), :]
    m_self = _ov_mask(cx1, cy1, cx2, cy2, ca, x1i, y1i, x2i, y2i, ai) * lower

    def fix_cond(c):
      return c[1] > 0

    def fix_body(c):
      a, _ = c
      dead = lax.dot_general(a, m_self, (((1,), (0,)), ((), ())),
                             preferred_element_type=f32)
      a_new = prior * jnp.where(dead > 0.0, 0.0, 1.0)
      changed = jnp.any(a_new != a).astype(jnp.int32)
      return (a_new, changed)

    a_fix, _ = lax.while_loop(fix_cond, fix_body, (prior, jnp.int32(1)))
    keep_ref[pl.ds(i, 1), :] = a_fix

    def inner(j, carry2):
      r_x1 = x1_ref[pl.ds(j, 1), :]
      r_y1 = y1_ref[pl.ds(j, 1), :]
      r_x2 = x2_ref[pl.ds(j, 1), :]
      r_y2 = y2_ref[pl.ds(j, 1), :]
      r_a = area_ref[pl.ds(j, 1), :]
      m = _ov_mask(cx1, cy1, cx2, cy2, ca, r_x1, r_y1, r_x2, r_y2, r_a)
      contrib = lax.dot_general(a_fix, m, (((1,), (0,)), ((), ())),
                                preferred_element_type=f32)
      keep_ref[pl.ds(j, 1), :] = (
          keep_ref[pl.ds(j, 1), :] * jnp.where(contrib > 0.0, 0.0, 1.0))
      return carry2

    lax.fori_loop(i + 1, _NB, inner, 0)
    return carry

  lax.fori_loop(0, _NB, outer, 0)


def _nms_keep(x1, y1, x2, y2, area):
  return pl.pallas_call(
      _nms_kernel,
      out_shape=jax.ShapeDtypeStruct((_NB, _B), jnp.float32),
  )(x1, y1, x2, y2, area)


def kernel(anchors, rpn_cls_prob, rpn_bbox_pred, rpn_trans_param, im_info):
  f32 = jnp.float32
  scores = rpn_cls_prob[0, :, :, _NUM_ANCHORS:].reshape(-1)
  deltas = rpn_bbox_pred.reshape(-1, 4)
  trans = rpn_trans_param.reshape(-1, 6)

  scores_sorted, order = lax.top_k(scores, _PRE)
  anch = jnp.take(anchors, order, axis=0)
  dels = jnp.take(deltas, order, axis=0)

  widths = anch[:, 2] - anch[:, 0] + 1.0
  heights = anch[:, 3] - anch[:, 1] + 1.0
  ctr_x = anch[:, 0] + 0.5 * widths
  ctr_y = anch[:, 1] + 0.5 * heights
  pred_ctr_x = dels[:, 0] * widths + ctr_x
  pred_ctr_y = dels[:, 1] * heights + ctr_y
  pred_w = jnp.exp(dels[:, 2]) * widths
  pred_h = jnp.exp(dels[:, 3]) * heights
  x1 = jnp.clip(pred_ctr_x - 0.5 * pred_w, 0.0, im_info[1] - 1.0)
  y1 = jnp.clip(pred_ctr_y - 0.5 * pred_h, 0.0, im_info[0] - 1.0)
  x2 = jnp.clip(pred_ctr_x + 0.5 * pred_w, 0.0, im_info[1] - 1.0)
  y2 = jnp.clip(pred_ctr_y + 0.5 * pred_h, 0.0, im_info[0] - 1.0)
  props = jnp.stack([x1, y1, x2, y2], axis=1)

  pad = _NB * _B - _PRE
  padv = jnp.zeros((pad,), f32)
  x1p = jnp.concatenate([x1, padv]).reshape(_NB, _B)
  y1p = jnp.concatenate([y1, padv]).reshape(_NB, _B)
  x2p = jnp.concatenate([x2, padv]).reshape(_NB, _B)
  y2p = jnp.concatenate([y2, padv]).reshape(_NB, _B)
  areap = (x2p - x1p + 1.0) * (y2p - y1p + 1.0)

  keep = _nms_keep(x1p, y1p, x2p, y2p, areap).reshape(-1)[:_PRE]

  idx = jnp.arange(_PRE, dtype=jnp.int32)
  rank = jnp.where(keep > 0.5, idx, _PRE + idx)
  sel = jnp.argsort(rank)[:_POST]

  props_k = jnp.take(props, sel, axis=0)
  scores_k = jnp.take(scores_sorted, sel)
  trans_k = jnp.take(trans, jnp.take(order, sel), axis=0)
  blob = jnp.concatenate([jnp.zeros((_POST, 1), f32), props_k], axis=1)
  return (blob, scores_k, trans_k)
